# baseline (speedup 1.0000x reference)
import functools

import jax
import jax.numpy as jnp
from jax import lax
from jax.experimental import pallas as pl
from jax.experimental.pallas import tpu as pltpu
from jax.experimental.pallas import tpu_sc as plsc

DIM = 64
NC, NS = 2, 16
NW = NC * NS
BATCH = 4096
HIST = 200
B = BATCH * HIST
BPW = B // NW
BB = BATCH // NW

_mesh = plsc.VectorSubcoreMesh(
    core_axis_name="c", subcore_axis_name="s", num_cores=NC, num_subcores=NS)


@functools.partial(
    pl.kernel,
    out_type=jax.ShapeDtypeStruct((HIST, 8, NW, 8, 128), jnp.float32),
    mesh=_mesh,
    scratch_types=[
        pltpu.VMEM((BPW,), jnp.int32),
        pltpu.VMEM((BB,), jnp.int32),
        pltpu.VMEM((BB,), jnp.int32),
        pltpu.VMEM((BB, DIM), jnp.float32),
        pltpu.VMEM((BB, DIM), jnp.float32),
        pltpu.VMEM((8, 1, 8, 128), jnp.float32),
        pltpu.VMEM((8, 1, 8, 128), jnp.float32),
        pltpu.SemaphoreType.DMA,
        pltpu.SemaphoreType.DMA,
        pltpu.SemaphoreType.DMA,
        pltpu.SemaphoreType.DMA,
    ],
    compiler_params=pltpu.CompilerParams(
        use_tc_tiling_on_sc=False, needs_layout_passes=False),
)
def _gather(idx_hbm, tab_hbm, out_hbm, idx_v, idxc0, idxc1, rows0, rows1,
            tbuf0, tbuf1, gsem0, gsem1, ssem0, ssem1):
    wid = lax.axis_index("s") * NC + lax.axis_index("c")
    pltpu.sync_copy(idx_hbm.at[pl.ds(wid * BPW, BPW)], idx_v)

    ii = lax.iota(jnp.int32, 16)
    row16 = [ii + j * 16 for j in range(8)]
    pos16 = [(ii + j * 16) * HIST for j in range(8)]

    def extract(h, idxc):
        for j in range(8):
            v = plsc.load_gather(idx_v, [pos16[j] + h])
            idxc[pl.ds(j * 16, 16)] = v

    def transpose(rows, tbuf):
        for d in range(DIM):
            for j in range(8):
                v = plsc.load_gather(rows, [row16[j], jnp.full((16,), d, jnp.int32)])
                tbuf[d // 8, 0, d % 8, pl.ds(j * 16, 16)] = v

    def out_sl(h):
        return out_hbm.at[h, pl.ds(0, 8), pl.ds(wid, 1), pl.ds(0, 8), pl.ds(0, 128)]

    def gather(idxc, rows, sem):
        return pltpu.async_copy(tab_hbm.at[idxc], rows, sem)

    extract(0, idxc0)
    gather(idxc0, rows0, gsem0)

    @pl.loop(0, HIST, step=2)
    def _pair(h):
        extract(h + 1, idxc1)
        gather(idxc1, rows1, gsem1)
        pltpu.make_async_copy(tab_hbm.at[idxc0], rows0, gsem0).wait()
        transpose(rows0, tbuf0)

        @pl.when(h > 0)
        def _():
            pltpu.make_async_copy(tbuf1, out_sl(h - 1), ssem1).wait()

        pltpu.async_copy(tbuf0, out_sl(h), ssem0)

        @pl.when(h + 2 < HIST)
        def _():
            extract(h + 2, idxc0)
            gather(idxc0, rows0, gsem0)

        pltpu.make_async_copy(tab_hbm.at[idxc1], rows1, gsem1).wait()
        transpose(rows1, tbuf1)
        pltpu.make_async_copy(tbuf0, out_sl(h), ssem0).wait()
        pltpu.async_copy(tbuf1, out_sl(h + 1), ssem1)

    pltpu.make_async_copy(tbuf1, out_sl(HIST - 1), ssem1).wait()


def kernel(indices, weight):
    out = _gather(indices.reshape(B), weight)
    return out.transpose(2, 4, 0, 1, 3).reshape(BATCH, HIST, DIM)


# speedup vs baseline: 1.2073x; 1.2073x over previous
import functools

import jax
import jax.numpy as jnp
from jax import lax
from jax.experimental import pallas as pl
from jax.experimental.pallas import tpu as pltpu
from jax.experimental.pallas import tpu_sc as plsc

DIM = 64
NC, NS = 2, 16
NW = NC * NS
BATCH = 4096
HIST = 200
B = BATCH * HIST
BPW = B // NW
BB = BATCH // NW

_mesh = plsc.VectorSubcoreMesh(
    core_axis_name="c", subcore_axis_name="s", num_cores=NC, num_subcores=NS)


@functools.partial(
    pl.kernel,
    out_type=jax.ShapeDtypeStruct((HIST, 8, NW, 8, 128), jnp.float32),
    mesh=_mesh,
    scratch_types=[
        pltpu.VMEM((BB,), jnp.int32),
        pltpu.VMEM((BB,), jnp.int32),
        pltpu.VMEM((BB, DIM), jnp.float32),
        pltpu.VMEM((BB, DIM), jnp.float32),
        pltpu.VMEM((8, 1, 8, 128), jnp.float32),
        pltpu.VMEM((8, 1, 8, 128), jnp.float32),
        pltpu.SemaphoreType.DMA,
        pltpu.SemaphoreType.DMA,
        pltpu.SemaphoreType.DMA,
        pltpu.SemaphoreType.DMA,
    ],
    compiler_params=pltpu.CompilerParams(
        use_tc_tiling_on_sc=False, needs_layout_passes=False),
)
def _gather(idx_hbm, tab_hbm, out_hbm, idxc0, idxc1, rows0, rows1,
            tbuf0, tbuf1, gsem0, gsem1, ssem0, ssem1):
    wid = lax.axis_index("s") * NC + lax.axis_index("c")

    ii = lax.iota(jnp.int32, 16)
    d8c = [(ii + k * 16) >> 3 for k in range(4)]
    dl8c = [(ii + k * 16) & 7 for k in range(4)]
    zero16 = ii & 0

    def extract(h, idxc):
        pltpu.sync_copy(idx_hbm.at[pl.ds(h * BATCH + wid * BB, BB)], idxc)

    def transpose(rows, tbuf):
        for b0 in range(0, BB, 4):
            vs = [rows[b0 + t, pl.ds(k * 16, 16)]
                  for t in range(4) for k in range(4)]
            for t in range(4):
                bvec = zero16 + (b0 + t)
                for k in range(4):
                    plsc.store_scatter(
                        tbuf, [d8c[k], zero16, dl8c[k], bvec], vs[t * 4 + k])

    def out_sl(h):
        return out_hbm.at[h, pl.ds(0, 8), pl.ds(wid, 1), pl.ds(0, 8), pl.ds(0, 128)]

    def gather(idxc, rows, sem):
        return pltpu.async_copy(tab_hbm.at[idxc], rows, sem)

    extract(jnp.int32(0), idxc0)
    gather(idxc0, rows0, gsem0)

    @pl.loop(0, HIST, step=2)
    def _pair(h):
        extract(h + 1, idxc1)
        gather(idxc1, rows1, gsem1)
        pltpu.make_async_copy(tab_hbm.at[idxc0], rows0, gsem0).wait()
        transpose(rows0, tbuf0)

        @pl.when(h > 0)
        def _():
            pltpu.make_async_copy(tbuf1, out_sl(h - 1), ssem1).wait()

        pltpu.async_copy(tbuf0, out_sl(h), ssem0)

        @pl.when(h + 2 < HIST)
        def _():
            extract(h + 2, idxc0)
            gather(idxc0, rows0, gsem0)

        pltpu.make_async_copy(tab_hbm.at[idxc1], rows1, gsem1).wait()
        transpose(rows1, tbuf1)
        pltpu.make_async_copy(tbuf0, out_sl(h), ssem0).wait()
        pltpu.async_copy(tbuf1, out_sl(h + 1), ssem1)

    pltpu.make_async_copy(tbuf1, out_sl(HIST - 1), ssem1).wait()


def kernel(indices, weight):
    out = _gather(indices.T.reshape(B), weight)
    return out.transpose(2, 4, 0, 1, 3).reshape(BATCH, HIST, DIM)


# speedup vs baseline: 1.4417x; 1.1941x over previous
import functools

import jax
import jax.numpy as jnp
from jax import lax
from jax.experimental import pallas as pl
from jax.experimental.pallas import tpu as pltpu
from jax.experimental.pallas import tpu_sc as plsc

DIM = 64
NC, NS = 2, 16
NW = NC * NS
BATCH = 4096
HIST = 200
B = BATCH * HIST
BPW = B // NW
BB = BATCH // NW

_mesh = plsc.VectorSubcoreMesh(
    core_axis_name="c", subcore_axis_name="s", num_cores=NC, num_subcores=NS)


@functools.partial(
    pl.kernel,
    out_type=jax.ShapeDtypeStruct((HIST, 8, NW, 8, 128), jnp.float32),
    mesh=_mesh,
    scratch_types=[
        pltpu.VMEM((BB,), jnp.int32),
        pltpu.VMEM((BB,), jnp.int32),
        pltpu.VMEM((BB, DIM), jnp.float32),
        pltpu.VMEM((BB, DIM), jnp.float32),
        pltpu.VMEM((8, 1, 8, 128), jnp.float32),
        pltpu.VMEM((8, 1, 8, 128), jnp.float32),
        pltpu.SemaphoreType.DMA,
        pltpu.SemaphoreType.DMA,
        pltpu.SemaphoreType.DMA,
        pltpu.SemaphoreType.DMA,
    ],
    compiler_params=pltpu.CompilerParams(
        use_tc_tiling_on_sc=False, needs_layout_passes=False),
)
def _gather(idx_hbm, tab_hbm, out_hbm, idxc0, idxc1, rows0, rows1,
            tbuf0, tbuf1, gsem0, gsem1, ssem0, ssem1):
    wid = lax.axis_index("s") * NC + lax.axis_index("c")

    ii = lax.iota(jnp.int32, 16)
    d8c = [(ii + k * 16) >> 3 for k in range(4)]
    dl8c = [(ii + k * 16) & 7 for k in range(4)]
    zero16 = ii & 0

    def extract(h, idxc):
        pltpu.sync_copy(idx_hbm.at[pl.ds(h * BATCH + wid * BB, BB)], idxc)

    def transpose(rows, tbuf):
        @plsc.parallel_loop(0, BB, step=1, unroll=8)
        def _b(b):
            bvec = zero16 + b
            for k in range(4):
                v = rows[b, pl.ds(k * 16, 16)]
                plsc.store_scatter(tbuf, [d8c[k], zero16, dl8c[k], bvec], v)

    def out_sl(h):
        return out_hbm.at[h, pl.ds(0, 8), pl.ds(wid, 1), pl.ds(0, 8), pl.ds(0, 128)]

    def gather(idxc, rows, sem):
        return pltpu.async_copy(tab_hbm.at[idxc], rows, sem)

    extract(jnp.int32(0), idxc0)
    gather(idxc0, rows0, gsem0)

    @pl.loop(0, HIST, step=2)
    def _pair(h):
        extract(h + 1, idxc1)
        gather(idxc1, rows1, gsem1)
        pltpu.make_async_copy(tab_hbm.at[idxc0], rows0, gsem0).wait()
        transpose(rows0, tbuf0)

        @pl.when(h > 0)
        def _():
            pltpu.make_async_copy(tbuf1, out_sl(h - 1), ssem1).wait()

        pltpu.async_copy(tbuf0, out_sl(h), ssem0)

        @pl.when(h + 2 < HIST)
        def _():
            extract(h + 2, idxc0)
            gather(idxc0, rows0, gsem0)

        pltpu.make_async_copy(tab_hbm.at[idxc1], rows1, gsem1).wait()
        transpose(rows1, tbuf1)
        pltpu.make_async_copy(tbuf0, out_sl(h), ssem0).wait()
        pltpu.async_copy(tbuf1, out_sl(h + 1), ssem1)

    pltpu.make_async_copy(tbuf1, out_sl(HIST - 1), ssem1).wait()


def kernel(indices, weight):
    out = _gather(indices.T.reshape(B), weight)
    return out.transpose(2, 4, 0, 1, 3).reshape(BATCH, HIST, DIM)


# speedup vs baseline: 2.4299x; 1.6855x over previous
import functools

import jax
import jax.numpy as jnp
from jax import lax
from jax.experimental import pallas as pl
from jax.experimental.pallas import tpu as pltpu
from jax.experimental.pallas import tpu_sc as plsc

DIM = 64
NC, NS = 2, 16
NW = NC * NS
BATCH = 4096
HIST = 200
B = BATCH * HIST
BPW = B // NW
BB = BATCH // NW

_mesh = plsc.VectorSubcoreMesh(
    core_axis_name="c", subcore_axis_name="s", num_cores=NC, num_subcores=NS)


@functools.partial(
    pl.kernel,
    out_type=jax.ShapeDtypeStruct((HIST, 8, NW, 8, 128), jnp.float32),
    mesh=_mesh,
    scratch_types=[
        pltpu.VMEM((BB,), jnp.int32),
        pltpu.VMEM((BB,), jnp.int32),
        pltpu.VMEM((BB, DIM), jnp.float32),
        pltpu.VMEM((BB, DIM), jnp.float32),
        pltpu.VMEM((8, 1, 8, 128), jnp.float32),
        pltpu.VMEM((8, 1, 8, 128), jnp.float32),
        pltpu.SemaphoreType.DMA,
        pltpu.SemaphoreType.DMA,
        pltpu.SemaphoreType.DMA,
        pltpu.SemaphoreType.DMA,
    ],
    compiler_params=pltpu.CompilerParams(
        use_tc_tiling_on_sc=False, needs_layout_passes=False),
)
def _gather(idx_hbm, tab_hbm, out_hbm, idxc0, idxc1, rows0, rows1,
            tbuf0, tbuf1, gsem0, gsem1, ssem0, ssem1):
    wid = lax.axis_index("s") * NC + lax.axis_index("c")

    ii = lax.iota(jnp.int32, 16)
    d8c = [(ii + k * 16) >> 3 for k in range(4)]
    dl8c = [(ii + k * 16) & 7 for k in range(4)]
    zero16 = ii & 0

    def extract(h, idxc):
        pltpu.sync_copy(idx_hbm.at[pl.ds(h * BATCH + wid * BB, BB)], idxc)

    def transpose(rows, tbuf):
        @plsc.parallel_loop(0, 64, step=1, unroll=2)
        def _t(x):
            dvec = ((x >> 4) << 4) + ((ii + x) & 15)
            d8v = dvec >> 3
            dl8v = dvec & 7
            for tb in range(8):
                rowv = ii + tb * 16
                v = plsc.load_gather(rows, [rowv, dvec])
                plsc.store_scatter(tbuf, [d8v, zero16, dl8v, rowv], v)

    def out_sl(h):
        return out_hbm.at[h, pl.ds(0, 8), pl.ds(wid, 1), pl.ds(0, 8), pl.ds(0, 128)]

    def gather(idxc, rows, sem):
        return pltpu.async_copy(tab_hbm.at[idxc], rows, sem)

    extract(jnp.int32(0), idxc0)
    gather(idxc0, rows0, gsem0)

    @pl.loop(0, HIST, step=2)
    def _pair(h):
        extract(h + 1, idxc1)
        gather(idxc1, rows1, gsem1)
        pltpu.make_async_copy(tab_hbm.at[idxc0], rows0, gsem0).wait()
        transpose(rows0, tbuf0)

        @pl.when(h > 0)
        def _():
            pltpu.make_async_copy(tbuf1, out_sl(h - 1), ssem1).wait()

        pltpu.async_copy(tbuf0, out_sl(h), ssem0)

        @pl.when(h + 2 < HIST)
        def _():
            extract(h + 2, idxc0)
            gather(idxc0, rows0, gsem0)

        pltpu.make_async_copy(tab_hbm.at[idxc1], rows1, gsem1).wait()
        transpose(rows1, tbuf1)
        pltpu.make_async_copy(tbuf0, out_sl(h), ssem0).wait()
        pltpu.async_copy(tbuf1, out_sl(h + 1), ssem1)

    pltpu.make_async_copy(tbuf1, out_sl(HIST - 1), ssem1).wait()


def kernel(indices, weight):
    out = _gather(indices.T.reshape(B), weight)
    return out.transpose(2, 4, 0, 1, 3).reshape(BATCH, HIST, DIM)


# speedup vs baseline: 4.3589x; 1.7939x over previous
import functools

import jax
import jax.numpy as jnp
from jax import lax
from jax.experimental import pallas as pl
from jax.experimental.pallas import tpu as pltpu
from jax.experimental.pallas import tpu_sc as plsc

DIM = 64
NC, NS = 2, 16
NW = NC * NS
BATCH = 4096
HIST = 200
B = BATCH * HIST
BPW = B // NW
BB = BATCH // NW

_mesh = plsc.VectorSubcoreMesh(
    core_axis_name="c", subcore_axis_name="s", num_cores=NC, num_subcores=NS)


@functools.partial(
    pl.kernel,
    out_type=jax.ShapeDtypeStruct((HIST, 8, NW, 8, 128), jnp.float32),
    mesh=_mesh,
    scratch_types=[
        pltpu.VMEM((BB,), jnp.int32),
        pltpu.VMEM((BB,), jnp.int32),
        pltpu.VMEM((BB, DIM), jnp.float32),
        pltpu.VMEM((BB, DIM), jnp.float32),
        pltpu.VMEM((8, 1, 8, 128), jnp.float32),
        pltpu.VMEM((8, 1, 8, 128), jnp.float32),
        pltpu.SemaphoreType.DMA,
        pltpu.SemaphoreType.DMA,
        pltpu.SemaphoreType.DMA,
        pltpu.SemaphoreType.DMA,
    ],
    compiler_params=pltpu.CompilerParams(
        use_tc_tiling_on_sc=False, needs_layout_passes=False),
)
def _gather(idx_hbm, tab_hbm, out_hbm, idxc0, idxc1, rows0, rows1,
            tbuf0, tbuf1, gsem0, gsem1, ssem0, ssem1):
    wid = lax.axis_index("s") * NC + lax.axis_index("c")

    ii = lax.iota(jnp.int32, 16)
    d8c = [(ii + k * 16) >> 3 for k in range(4)]
    dl8c = [(ii + k * 16) & 7 for k in range(4)]
    zero16 = ii & 0

    def extract(h, idxc):
        pltpu.sync_copy(idx_hbm.at[pl.ds(h * BATCH + wid * BB, BB)], idxc)

    def transpose(rows, tbuf):
        @plsc.parallel_loop(0, 64, step=1, unroll=2)
        def _t(x):
            dvec = ((x >> 4) << 4) + ((ii + x) & 15)
            d8v = dvec >> 3
            dl8v = dvec & 7
            for tb in range(8):
                rowv = ii + tb * 16
                v = plsc.load_gather(rows, [rowv, dvec])
                plsc.store_scatter(tbuf, [d8v, zero16, dl8v, rowv], v)

    def out_sl(h):
        return out_hbm.at[h, pl.ds(0, 8), pl.ds(wid, 1), pl.ds(0, 8), pl.ds(0, 128)]

    def gather(idxc, rows, sem):
        return pltpu.async_copy(tab_hbm.at[idxc], rows, sem)

    extract(jnp.int32(0), idxc0)
    gather(idxc0, rows0, gsem0)

    @pl.loop(0, HIST, step=2)
    def _pair(h):
        extract(h + 1, idxc1)
        gather(idxc1, rows1, gsem1)
        pltpu.make_async_copy(tab_hbm.at[idxc0], rows0, gsem0).wait()
        transpose(rows0, tbuf0)

        @pl.when(h > 0)
        def _():
            pltpu.make_async_copy(tbuf1, out_sl(h - 1), ssem1).wait()

        pltpu.async_copy(tbuf0, out_sl(h), ssem0)

        @pl.when(h + 2 < HIST)
        def _():
            extract(h + 2, idxc0)
            gather(idxc0, rows0, gsem0)

        pltpu.make_async_copy(tab_hbm.at[idxc1], rows1, gsem1).wait()
        transpose(rows1, tbuf1)
        pltpu.make_async_copy(tbuf0, out_sl(h), ssem0).wait()
        pltpu.async_copy(tbuf1, out_sl(h + 1), ssem1)

    pltpu.make_async_copy(tbuf1, out_sl(HIST - 1), ssem1).wait()


NTILE_FULL = 7812
NT_PAIRS = 244


@functools.partial(
    pl.kernel,
    out_type=jax.ShapeDtypeStruct((1000000 * DIM,), jnp.float32),
    mesh=_mesh,
    scratch_types=[
        pltpu.VMEM((DIM, 128), jnp.float32),
        pltpu.VMEM((DIM, 128), jnp.float32),
        pltpu.VMEM((8192,), jnp.float32),
        pltpu.VMEM((8192,), jnp.float32),
        pltpu.SemaphoreType.DMA,
        pltpu.SemaphoreType.DMA,
        pltpu.SemaphoreType.DMA,
        pltpu.SemaphoreType.DMA,
    ],
    compiler_params=pltpu.CompilerParams(needs_layout_passes=False),
)
def _detile(wt_hbm, out_hbm, s0, s1, t0, t1, lsem0, lsem1, wsem0, wsem1):
    wid = lax.axis_index("s") * NC + lax.axis_index("c")

    ii = lax.iota(jnp.int32, 16)
    dvs = [ii + d0 for d0 in range(0, DIM, 16)]

    def load(c, sbuf, sem):
        return pltpu.async_copy(
            wt_hbm.at[pl.ds(0, DIM), pl.ds(c * 128, 128)], sbuf, sem)

    def transpose(sbuf, tbuf):
        @plsc.parallel_loop(0, 128, step=1, unroll=2)
        def _t(x):
            ivec = ((x >> 4) << 4) + ((ii + x) & 15)
            iv64 = ivec << 6
            for k in range(4):
                v = plsc.load_gather(sbuf, [dvs[k], ivec])
                plsc.store_scatter(tbuf, [iv64 + dvs[k]], v)

    def store(c, tbuf, sem):
        return pltpu.async_copy(
            tbuf, out_hbm.at[pl.ds(c * 8192, 8192)], sem)

    load(wid, s0, lsem0)

    @pl.loop(0, NT_PAIRS, step=2)
    def _pair(k):
        c0 = wid + 32 * k
        c1 = c0 + 32
        load(c1, s1, lsem1)
        pltpu.make_async_copy(
            wt_hbm.at[pl.ds(0, DIM), pl.ds(c0 * 128, 128)], s0, lsem0).wait()
        transpose(s0, t0)

        @pl.when(k > 0)
        def _():
            pltpu.make_async_copy(
                t1, out_hbm.at[pl.ds((c0 - 32) * 8192, 8192)], wsem1).wait()

        store(c0, t0, wsem0)

        @pl.when(k + 2 < NT_PAIRS)
        def _():
            load(c0 + 64, s0, lsem0)

        pltpu.make_async_copy(
            wt_hbm.at[pl.ds(0, DIM), pl.ds(c1 * 128, 128)], s1, lsem1).wait()
        transpose(s1, t1)
        pltpu.make_async_copy(
            t0, out_hbm.at[pl.ds(c0 * 8192, 8192)], wsem0).wait()
        store(c1, t1, wsem1)

    pltpu.make_async_copy(
        t1, out_hbm.at[pl.ds((wid + 32 * (NT_PAIRS - 1)) * 8192, 8192)],
        wsem1).wait()

    c_ep = wid + 32 * NT_PAIRS

    @pl.when(wid <= 4)
    def _():
        pltpu.sync_copy(wt_hbm.at[pl.ds(0, DIM), pl.ds(c_ep * 128, 128)], s0)
        transpose(s0, t0)

        @pl.when(wid < 4)
        def _():
            pltpu.sync_copy(t0, out_hbm.at[pl.ds(c_ep * 8192, 8192)])

        @pl.when(wid == 4)
        def _():
            pltpu.sync_copy(t0.at[pl.ds(0, 4096)],
                            out_hbm.at[pl.ds(c_ep * 8192, 4096)])


def kernel(indices, weight):
    tab = _detile(weight.T).reshape(1000000, DIM)
    out = _gather(indices.T.reshape(B), tab)
    return out.transpose(2, 4, 0, 1, 3).reshape(BATCH, HIST, DIM)


# speedup vs baseline: 4.8858x; 1.1209x over previous
import functools

import jax
import jax.numpy as jnp
from jax import lax
from jax.experimental import pallas as pl
from jax.experimental.pallas import tpu as pltpu
from jax.experimental.pallas import tpu_sc as plsc

DIM = 64
NC, NS = 2, 16
NW = NC * NS
BATCH = 4096
HIST = 200
B = BATCH * HIST
BPW = B // NW
BB = BATCH // NW

_mesh = plsc.VectorSubcoreMesh(
    core_axis_name="c", subcore_axis_name="s", num_cores=NC, num_subcores=NS)


@functools.partial(
    pl.kernel,
    out_type=jax.ShapeDtypeStruct((HIST, 8, NW, 8, 128), jnp.float32),
    mesh=_mesh,
    scratch_types=[
        [pltpu.VMEM((BB,), jnp.int32)] * 4,
        [pltpu.SemaphoreType.DMA] * 4,
        pltpu.VMEM((BB, DIM), jnp.float32),
        pltpu.VMEM((BB, DIM), jnp.float32),
        pltpu.VMEM((8, 1, 8, 128), jnp.float32),
        pltpu.VMEM((8, 1, 8, 128), jnp.float32),
        pltpu.SemaphoreType.DMA,
        pltpu.SemaphoreType.DMA,
        pltpu.SemaphoreType.DMA,
        pltpu.SemaphoreType.DMA,
    ],
    compiler_params=pltpu.CompilerParams(
        use_tc_tiling_on_sc=False, needs_layout_passes=False),
)
def _gather(idx_hbm, tab_hbm, out_hbm, eb, esem, rows0, rows1,
            tbuf0, tbuf1, gsem0, gsem1, ssem0, ssem1):
    wid = lax.axis_index("s") * NC + lax.axis_index("c")

    ii = lax.iota(jnp.int32, 16)
    d8c = [(ii + k * 16) >> 3 for k in range(4)]
    dl8c = [(ii + k * 16) & 7 for k in range(4)]
    zero16 = ii & 0

    def ext_start(h, r):
        pltpu.async_copy(idx_hbm.at[pl.ds(h * BATCH + wid * BB, BB)],
                         eb[r], esem[r])

    def ext_wait(h, r):
        pltpu.make_async_copy(idx_hbm.at[pl.ds(h * BATCH + wid * BB, BB)],
                              eb[r], esem[r]).wait()

    def transpose(rows, tbuf):
        @plsc.parallel_loop(0, 64, step=1, unroll=2)
        def _t(x):
            dvec = ((x >> 4) << 4) + ((ii + x) & 15)
            d8v = dvec >> 3
            dl8v = dvec & 7
            for tb in range(8):
                rowv = ii + tb * 16
                v = plsc.load_gather(rows, [rowv, dvec])
                plsc.store_scatter(tbuf, [d8v, zero16, dl8v, rowv], v)

    def out_sl(h):
        return out_hbm.at[h, pl.ds(0, 8), pl.ds(wid, 1), pl.ds(0, 8), pl.ds(0, 128)]

    def gather(idxc, rows, sem):
        return pltpu.async_copy(tab_hbm.at[idxc], rows, sem)

    rows = [rows0, rows1]
    tbuf = [tbuf0, tbuf1]
    gsem = [gsem0, gsem1]
    ssem = [ssem0, ssem1]

    z = jnp.int32(0)
    for t in range(4):
        ext_start(z + t, t)
    ext_wait(z, 0)
    gather(eb[0], rows0, gsem0)

    @pl.loop(0, HIST, step=4)
    def _quad(h):
        for t in range(4):
            p = t & 1
            q = 1 - p
            if t < 3:
                ext_wait(h + t + 1, t + 1)
                gather(eb[t + 1], rows[q], gsem[q])
            else:
                @pl.when(h + 4 < HIST)
                def _():
                    ext_wait(h + 4, 0)
                    gather(eb[0], rows[q], gsem[q])

            pltpu.make_async_copy(tab_hbm.at[eb[t]], rows[p], gsem[p]).wait()

            @pl.when(h + 4 + t < HIST)
            def _(t=t):
                ext_start(h + 4 + t, t)

            @pl.when(h + t - 2 >= 0)
            def _(t=t, p=p):
                pltpu.make_async_copy(tbuf[p], out_sl(h + t - 2),
                                      ssem[p]).wait()

            transpose(rows[p], tbuf[p])
            pltpu.async_copy(tbuf[p], out_sl(h + t), ssem[p])

    pltpu.make_async_copy(tbuf0, out_sl(HIST - 2), ssem0).wait()
    pltpu.make_async_copy(tbuf1, out_sl(HIST - 1), ssem1).wait()


NTILE_FULL = 7812
NT_PAIRS = 244


@functools.partial(
    pl.kernel,
    out_type=jax.ShapeDtypeStruct((1000000 * DIM,), jnp.float32),
    mesh=_mesh,
    scratch_types=[
        pltpu.VMEM((DIM, 128), jnp.float32),
        pltpu.VMEM((DIM, 128), jnp.float32),
        pltpu.VMEM((8192,), jnp.float32),
        pltpu.VMEM((8192,), jnp.float32),
        pltpu.SemaphoreType.DMA,
        pltpu.SemaphoreType.DMA,
        pltpu.SemaphoreType.DMA,
        pltpu.SemaphoreType.DMA,
    ],
    compiler_params=pltpu.CompilerParams(needs_layout_passes=False),
)
def _detile(wt_hbm, out_hbm, s0, s1, t0, t1, lsem0, lsem1, wsem0, wsem1):
    wid = lax.axis_index("s") * NC + lax.axis_index("c")

    ii = lax.iota(jnp.int32, 16)
    dvs = [ii + d0 for d0 in range(0, DIM, 16)]

    def load(c, sbuf, sem):
        return pltpu.async_copy(
            wt_hbm.at[pl.ds(0, DIM), pl.ds(c * 128, 128)], sbuf, sem)

    def transpose(sbuf, tbuf):
        @plsc.parallel_loop(0, 128, step=1, unroll=2)
        def _t(x):
            ivec = ((x >> 4) << 4) + ((ii + x) & 15)
            iv64 = ivec << 6
            for k in range(4):
                v = plsc.load_gather(sbuf, [dvs[k], ivec])
                plsc.store_scatter(tbuf, [iv64 + dvs[k]], v)

    def store(c, tbuf, sem):
        return pltpu.async_copy(
            tbuf, out_hbm.at[pl.ds(c * 8192, 8192)], sem)

    load(wid, s0, lsem0)

    @pl.loop(0, NT_PAIRS, step=2)
    def _pair(k):
        c0 = wid + 32 * k
        c1 = c0 + 32
        load(c1, s1, lsem1)
        pltpu.make_async_copy(
            wt_hbm.at[pl.ds(0, DIM), pl.ds(c0 * 128, 128)], s0, lsem0).wait()
        transpose(s0, t0)

        @pl.when(k > 0)
        def _():
            pltpu.make_async_copy(
                t1, out_hbm.at[pl.ds((c0 - 32) * 8192, 8192)], wsem1).wait()

        store(c0, t0, wsem0)

        @pl.when(k + 2 < NT_PAIRS)
        def _():
            load(c0 + 64, s0, lsem0)

        pltpu.make_async_copy(
            wt_hbm.at[pl.ds(0, DIM), pl.ds(c1 * 128, 128)], s1, lsem1).wait()
        transpose(s1, t1)
        pltpu.make_async_copy(
            t0, out_hbm.at[pl.ds(c0 * 8192, 8192)], wsem0).wait()
        store(c1, t1, wsem1)

    pltpu.make_async_copy(
        t1, out_hbm.at[pl.ds((wid + 32 * (NT_PAIRS - 1)) * 8192, 8192)],
        wsem1).wait()

    c_ep = wid + 32 * NT_PAIRS

    @pl.when(wid <= 4)
    def _():
        pltpu.sync_copy(wt_hbm.at[pl.ds(0, DIM), pl.ds(c_ep * 128, 128)], s0)
        transpose(s0, t0)

        @pl.when(wid < 4)
        def _():
            pltpu.sync_copy(t0, out_hbm.at[pl.ds(c_ep * 8192, 8192)])

        @pl.when(wid == 4)
        def _():
            pltpu.sync_copy(t0.at[pl.ds(0, 4096)],
                            out_hbm.at[pl.ds(c_ep * 8192, 4096)])


def kernel(indices, weight):
    tab = _detile(weight.T).reshape(1000000, DIM)
    out = _gather(indices.T.reshape(B), tab)
    return out.transpose(2, 4, 0, 1, 3).reshape(BATCH, HIST, DIM)
